# baseline (device time: 12977 ns/iter reference)
import jax
import jax.numpy as jnp
from jax import lax
from jax.experimental import pallas as pl
from jax.experimental.pallas import tpu as pltpu

N_DEV = 32
GROUP = 8
N_GROUPS = N_DEV // GROUP


def _combine(vals, idxs):
    best_val = jnp.max(vals, axis=0)
    best_idx = jnp.min(
        jnp.where(vals == best_val[None, :], idxs, jnp.float32(1e9)), axis=0
    )
    return best_val, best_idx


def kernel(x):
    m_per, n = x.shape

    def body(x_ref, out_ref, g1_ref, g2_ref, xv_ref,
             send1, recv1, send2, recv2, copy_sem):
        my_pos = lax.axis_index("i")
        my_rank = lax.rem(my_pos, GROUP)
        group_base = my_pos - my_rank

        in_copy = pltpu.make_async_copy(x_ref, xv_ref, copy_sem)
        in_copy.start()

        barrier_sem = pltpu.get_barrier_semaphore()
        for k in range(1, GROUP):
            pl.semaphore_signal(
                barrier_sem, inc=1,
                device_id=(group_base + lax.rem(my_rank + k, GROUP),),
                device_id_type=pl.DeviceIdType.MESH,
            )
        for j in range(1, N_GROUPS):
            pl.semaphore_signal(
                barrier_sem, inc=1,
                device_id=(lax.rem(my_pos + GROUP * j, N_DEV),),
                device_id_type=pl.DeviceIdType.MESH,
            )
        in_copy.wait()

        xv = xv_ref[:, :]
        val = jnp.max(xv, axis=0)
        rows = lax.broadcasted_iota(jnp.int32, (m_per, n), 0)
        loc_idx = jnp.min(jnp.where(xv == val[None, :], rows, m_per), axis=0)
        gidx = (loc_idx + my_pos * m_per).astype(jnp.float32)

        g1_ref[0, 0, :] = val
        g1_ref[0, 1, :] = gidx

        pl.semaphore_wait(barrier_sem, GROUP - 1 + N_GROUPS - 1)

        rdmas1 = []
        for k in range(1, GROUP):
            tgt = group_base + lax.rem(my_rank + k, GROUP)
            rdma = pltpu.make_async_remote_copy(
                src_ref=g1_ref.at[0],
                dst_ref=g1_ref.at[k],
                send_sem=send1.at[k],
                recv_sem=recv1.at[k],
                device_id=(tgt,),
                device_id_type=pl.DeviceIdType.MESH,
            )
            rdma.start()
            rdmas1.append(rdma)
        for rdma in rdmas1:
            rdma.wait_recv()

        gval, gidx2 = _combine(g1_ref[:, 0, :], g1_ref[:, 1, :])
        g2_ref[0, 0, :] = gval
        g2_ref[0, 1, :] = gidx2

        rdmas2 = []
        for j in range(1, N_GROUPS):
            tgt = lax.rem(my_pos + GROUP * j, N_DEV)
            rdma = pltpu.make_async_remote_copy(
                src_ref=g2_ref.at[0],
                dst_ref=g2_ref.at[j],
                send_sem=send2.at[j],
                recv_sem=recv2.at[j],
                device_id=(tgt,),
                device_id_type=pl.DeviceIdType.MESH,
            )
            rdma.start()
            rdmas2.append(rdma)
        for rdma in rdmas2:
            rdma.wait_recv()

        best_val, best_idx = _combine(g2_ref[:, 0, :], g2_ref[:, 1, :])
        out_ref[0, :] = best_val
        out_ref[1, :] = best_idx

        for rdma in rdmas1:
            rdma.wait_send()
        for rdma in rdmas2:
            rdma.wait_send()

    out_shape = jax.ShapeDtypeStruct((2, n), jnp.float32)
    return pl.pallas_call(
        body,
        out_shape=out_shape,
        in_specs=[pl.BlockSpec(memory_space=pl.ANY)],
        out_specs=pl.BlockSpec(memory_space=pltpu.VMEM),
        scratch_shapes=[
            pltpu.VMEM((GROUP, 2, n), jnp.float32),
            pltpu.VMEM((N_GROUPS, 2, n), jnp.float32),
            pltpu.VMEM((m_per, n), jnp.float32),
            pltpu.SemaphoreType.DMA((GROUP,)),
            pltpu.SemaphoreType.DMA((GROUP,)),
            pltpu.SemaphoreType.DMA((N_GROUPS,)),
            pltpu.SemaphoreType.DMA((N_GROUPS,)),
            pltpu.SemaphoreType.DMA,
        ],
        compiler_params=pltpu.CompilerParams(collective_id=0),
    )(x)


# device time: 12972 ns/iter; 1.0004x vs baseline; 1.0004x over previous
import jax
import jax.numpy as jnp
from jax import lax
from jax.experimental import pallas as pl
from jax.experimental.pallas import tpu as pltpu

N_DEV = 32
GROUP = 8
N_GROUPS = N_DEV // GROUP


def _combine(vals, idxs):
    best_val = jnp.max(vals, axis=0)
    best_idx = jnp.min(
        jnp.where(vals == best_val[None, :], idxs, jnp.float32(1e9)), axis=0
    )
    return best_val, best_idx


def kernel(x):
    m_per, n = x.shape

    def body(x_ref, out_ref, g1_ref, g2_ref, xv_ref, res_ref,
             send1, recv1, send2, recv2, copy_sem, out_sem):
        my_pos = lax.axis_index("i")
        my_rank = lax.rem(my_pos, GROUP)
        group_base = my_pos - my_rank

        in_copy = pltpu.make_async_copy(x_ref, xv_ref, copy_sem)
        in_copy.start()

        barrier_sem = pltpu.get_barrier_semaphore()
        for k in range(1, GROUP):
            pl.semaphore_signal(
                barrier_sem, inc=1,
                device_id=(group_base + lax.rem(my_rank + k, GROUP),),
                device_id_type=pl.DeviceIdType.MESH,
            )
        for j in range(1, N_GROUPS):
            pl.semaphore_signal(
                barrier_sem, inc=1,
                device_id=(lax.rem(my_pos + GROUP * j, N_DEV),),
                device_id_type=pl.DeviceIdType.MESH,
            )
        in_copy.wait()

        xv = xv_ref[:, :]
        val = jnp.max(xv, axis=0)
        rows = lax.broadcasted_iota(jnp.int32, (m_per, n), 0)
        loc_idx = jnp.min(jnp.where(xv == val[None, :], rows, m_per), axis=0)
        gidx = (loc_idx + my_pos * m_per).astype(jnp.float32)

        g1_ref[0, 0, :] = val
        g1_ref[0, 1, :] = gidx

        pl.semaphore_wait(barrier_sem, GROUP - 1 + N_GROUPS - 1)

        rdmas1 = []
        for k in range(1, GROUP):
            tgt = group_base + lax.rem(my_rank + k, GROUP)
            rdma = pltpu.make_async_remote_copy(
                src_ref=g1_ref.at[0],
                dst_ref=g1_ref.at[k],
                send_sem=send1.at[k],
                recv_sem=recv1.at[k],
                device_id=(tgt,),
                device_id_type=pl.DeviceIdType.MESH,
            )
            rdma.start()
            rdmas1.append(rdma)
        for rdma in rdmas1:
            rdma.wait_recv()

        gval, gidx2 = _combine(g1_ref[:, 0, :], g1_ref[:, 1, :])
        g2_ref[0, 0, :] = gval
        g2_ref[0, 1, :] = gidx2

        rdmas2 = []
        for j in range(1, N_GROUPS):
            tgt = lax.rem(my_pos + GROUP * j, N_DEV)
            rdma = pltpu.make_async_remote_copy(
                src_ref=g2_ref.at[0],
                dst_ref=g2_ref.at[j],
                send_sem=send2.at[j],
                recv_sem=recv2.at[j],
                device_id=(tgt,),
                device_id_type=pl.DeviceIdType.MESH,
            )
            rdma.start()
            rdmas2.append(rdma)
        for rdma in rdmas2:
            rdma.wait_recv()

        best_val, best_idx = _combine(g2_ref[:, 0, :], g2_ref[:, 1, :])
        res_ref[0, :] = best_val
        res_ref[1, :] = best_idx
        out_copy = pltpu.make_async_copy(res_ref, out_ref, out_sem)
        out_copy.start()
        out_copy.wait()

        for rdma in rdmas1:
            rdma.wait_send()
        for rdma in rdmas2:
            rdma.wait_send()

    out_shape = jax.ShapeDtypeStruct((2, n), jnp.float32)
    return pl.pallas_call(
        body,
        out_shape=out_shape,
        in_specs=[pl.BlockSpec(memory_space=pl.ANY)],
        out_specs=pl.BlockSpec(memory_space=pl.ANY),
        scratch_shapes=[
            pltpu.VMEM((GROUP, 2, n), jnp.float32),
            pltpu.VMEM((N_GROUPS, 2, n), jnp.float32),
            pltpu.VMEM((m_per, n), jnp.float32),
            pltpu.VMEM((2, n), jnp.float32),
            pltpu.SemaphoreType.DMA((GROUP,)),
            pltpu.SemaphoreType.DMA((GROUP,)),
            pltpu.SemaphoreType.DMA((N_GROUPS,)),
            pltpu.SemaphoreType.DMA((N_GROUPS,)),
            pltpu.SemaphoreType.DMA,
            pltpu.SemaphoreType.DMA,
        ],
        compiler_params=pltpu.CompilerParams(collective_id=0),
    )(x)
